# baseline (speedup 1.0000x reference)
import functools

import jax
import jax.numpy as jnp
from jax import lax
from jax.experimental import pallas as pl
from jax.experimental.pallas import tpu as pltpu
from jax.experimental.pallas import tpu_sc as plsc

N = 10000
E = 320000
D = 128
H = 64
G = 64

NP = 10240
RPT = NP // 16
NE = E + N
NB = 4
SCH = 164
CH = SCH + NB
EPT = CH * 128
EP = 16 * EPT

_mesh = plsc.VectorSubcoreMesh(core_axis_name="c", subcore_axis_name="s")
_sc_params = pltpu.CompilerParams(use_tc_tiling_on_sc=False)


@functools.partial(
    pl.kernel,
    out_type=jax.ShapeDtypeStruct((2, NP), jnp.float32),
    mesh=_mesh,
    scratch_types=[
        pltpu.VMEM((CH, 128), jnp.int32),
        pltpu.VMEM((128,), jnp.float32),
        pltpu.VMEM((RPT,), jnp.float32),
        pltpu.VMEM_SHARED((NP,), jnp.float32),
    ],
    compiler_params=_sc_params,
)
def _sc_deg(dst_hbm, deg_out, dstbuf, ones, zbuf, deg_sh):
    c = lax.axis_index("c")
    s = lax.axis_index("s")

    @pl.loop(0, 128, step=16)
    def _(i):
        ones[pl.ds(i, 16)] = jnp.full((16,), 1.0, jnp.float32)

    @pl.loop(0, RPT, step=16)
    def _(i):
        zbuf[pl.ds(i, 16)] = jnp.zeros((16,), jnp.float32)

    pltpu.sync_copy(zbuf, deg_sh.at[pl.ds(s * RPT, RPT)])
    plsc.subcore_barrier()

    pltpu.sync_copy(dst_hbm.at[c, s], dstbuf)

    @pl.loop(0, CH)
    def _(j):
        pltpu.sync_copy(ones, deg_sh.at[dstbuf.at[j]], add=True)

    plsc.subcore_barrier()
    pltpu.sync_copy(deg_sh.at[pl.ds(s * RPT, RPT)],
                    deg_out.at[c, pl.ds(s * RPT, RPT)])


@functools.partial(
    pl.kernel,
    out_type=jax.ShapeDtypeStruct((2, NP, H), jnp.float32),
    mesh=_mesh,
    scratch_types=[
        pltpu.VMEM((CH, 128), jnp.int32),
        pltpu.VMEM((CH, 128), jnp.int32),
        [pltpu.VMEM((128, H), jnp.float32) for _ in range(NB)],
        [pltpu.SemaphoreType.DMA for _ in range(NB)],
        [pltpu.SemaphoreType.DMA for _ in range(NB)],
        pltpu.VMEM_SHARED((NP, H), jnp.float32),
    ],
    compiler_params=_sc_params,
)
def _sc_agg(scaled_hbm, src_hbm, dst_hbm, acc_out,
            srcbuf, dstbuf, bufs, gsems, ssems, acc_sh):
    c = lax.axis_index("c")
    s = lax.axis_index("s")

    @pl.loop(0, 128)
    def _(r):
        @pl.loop(0, H, step=16)
        def _(k):
            bufs[0][r, pl.ds(k, 16)] = jnp.zeros((16,), jnp.float32)

    @pl.loop(0, RPT // 128)
    def _(k):
        pltpu.sync_copy(bufs[0], acc_sh.at[pl.ds(s * RPT + k * 128, 128)])

    plsc.subcore_barrier()

    pltpu.sync_copy(src_hbm.at[c, s], srcbuf)
    pltpu.sync_copy(dst_hbm.at[c, s], dstbuf)

    def gather(j, k):
        pltpu.async_copy(scaled_hbm.at[srcbuf.at[j]], bufs[k], gsems[k])

    def gwait(j, k):
        pltpu.make_async_copy(scaled_hbm.at[srcbuf.at[j]], bufs[k],
                              gsems[k]).wait()

    def scat(j, k):
        pltpu.async_copy(bufs[k], acc_sh.at[dstbuf.at[j]], ssems[k], add=True)

    def swait(j, k):
        pltpu.make_async_copy(bufs[k], acc_sh.at[dstbuf.at[j]],
                              ssems[k]).wait()

    for k in range(NB):
        gather(k, k)

    @pl.loop(0, SCH, step=NB)
    def _(j):
        for k in range(NB):
            gwait(j + k, k)
            scat(j + k, k)
        for k in range(NB):
            swait(j + k, k)
            gather(j + NB + k, k)

    for k in range(NB):
        gwait(SCH + k, k)

    plsc.subcore_barrier()

    @pl.loop(0, RPT // 128)
    def _(k):
        off = s * RPT + k * 128
        pltpu.sync_copy(acc_sh.at[pl.ds(off, 128)],
                        acc_out.at[c, pl.ds(off, 128)])


def _tc_scale_body(x_ref, w_ref, deg_ref, o_ref):
    h = jnp.dot(x_ref[...], w_ref[...], preferred_element_type=jnp.float32)
    dinv = lax.rsqrt(jnp.maximum(deg_ref[...], 1.0))
    o_ref[...] = h * dinv


def _tc_scale(xcat, w, degcol):
    blk = 1024
    grid = (2 * NP // blk,)
    return pl.pallas_call(
        _tc_scale_body,
        grid=grid,
        in_specs=[
            pl.BlockSpec((blk, D), lambda i: (i, 0)),
            pl.BlockSpec((D, H), lambda i: (0, 0)),
            pl.BlockSpec((blk, 1), lambda i: (i, 0)),
        ],
        out_specs=pl.BlockSpec((blk, H), lambda i: (i, 0)),
        out_shape=jax.ShapeDtypeStruct((2 * NP, H), jnp.float32),
    )(xcat, w, degcol)


def _tc_head_body(acc_ref, deg_ref, batch_ref, bg_ref,
                  w1_ref, b1_ref, w2_ref, b2_ref, w3_ref, b3_ref,
                  w4_ref, b4_ref, o_ref):
    means = []
    for g in range(2):
        accg = acc_ref[g]
        degg = deg_ref[g]
        dinv = lax.rsqrt(jnp.maximum(degg, 1.0))
        outg = jnp.maximum(accg * dinv + bg_ref[...], 0.0)
        batchg = batch_ref[g]
        iot = lax.broadcasted_iota(jnp.int32, (G, NP), 0)
        oh = (iot == batchg).astype(jnp.float32)
        sums = jnp.dot(oh, outg, preferred_element_type=jnp.float32)
        cnts = jnp.sum(oh, axis=1, keepdims=True)
        means.append(sums / jnp.maximum(cnts, 1.0))
    z = jnp.concatenate(means, axis=1)
    z = jnp.maximum(jnp.dot(z, w1_ref[...],
                            preferred_element_type=jnp.float32) + b1_ref[...], 0.0)
    z = jnp.maximum(jnp.dot(z, w2_ref[...],
                            preferred_element_type=jnp.float32) + b2_ref[...], 0.0)
    z = jnp.maximum(jnp.dot(z, w3_ref[...],
                            preferred_element_type=jnp.float32) + b3_ref[...], 0.0)
    z = jnp.dot(z, w4_ref[...], preferred_element_type=jnp.float32) + b4_ref[...]
    o_ref[...] = 1.0 / (1.0 + jnp.exp(-z))


def _tc_head(acc3, deg3, batch3, bg, w1, b1, w2, b2, w3, b3, w4, b4):
    return pl.pallas_call(
        _tc_head_body,
        out_shape=jax.ShapeDtypeStruct((G, 1), jnp.float32),
    )(acc3, deg3, batch3, bg, w1, b1, w2, b2, w3, b3, w4, b4)


def kernel(x_1, edge_index_1, x_1_batch, x_2, edge_index_2, x_2_batch,
           W_gcn, b_gcn, l1_w, l1_b, l2_w, l2_b, l3_w, l3_b, l4_w, l4_b):
    loop = jnp.arange(N, dtype=jnp.int32)

    def prep(ei, g):
        src = jnp.concatenate([ei[0], loop])
        dst = jnp.concatenate([ei[1], loop])
        src = jnp.pad(src, (0, EP - NE)) + g * NP
        dst = jnp.pad(dst, (0, EP - NE), constant_values=N)
        return src.reshape(16, CH, 128), dst.reshape(16, CH, 128)

    s1, d1 = prep(edge_index_1, 0)
    s2, d2 = prep(edge_index_2, 1)
    srccat = jnp.stack([s1, s2])
    dstcat = jnp.stack([d1, d2])

    xcat = jnp.concatenate([
        jnp.pad(x_1, ((0, NP - N), (0, 0))),
        jnp.pad(x_2, ((0, NP - N), (0, 0))),
    ])

    deg = _sc_deg(dstcat)
    scaled = _tc_scale(xcat, W_gcn, deg.reshape(2 * NP, 1))
    acc = _sc_agg(scaled, srccat, dstcat)

    batchcat = jnp.stack([
        jnp.pad(x_1_batch, (0, NP - N), constant_values=G),
        jnp.pad(x_2_batch, (0, NP - N), constant_values=G),
    ]).reshape(2, 1, NP)

    return _tc_head(
        acc, deg.reshape(2, NP, 1), batchcat,
        b_gcn.reshape(1, H),
        l1_w, l1_b.reshape(1, 64),
        l2_w, l2_b.reshape(1, 32),
        l3_w, l3_b.reshape(1, 16),
        l4_w, l4_b.reshape(1, 1),
    )


# speedup vs baseline: 1.4591x; 1.4591x over previous
import functools

import jax
import jax.numpy as jnp
from jax import lax
from jax.experimental import pallas as pl
from jax.experimental.pallas import tpu as pltpu
from jax.experimental.pallas import tpu_sc as plsc

N = 10000
E = 320000
D = 128
H = 64
G = 64

NP = 10240
RPT = NP // 16
NE = E + N
NB = 2
SCH = 162
CH = SCH + NB
EPT = CH * 128
EP = 16 * EPT

_mesh = plsc.VectorSubcoreMesh(core_axis_name="c", subcore_axis_name="s")
_sc_params = pltpu.CompilerParams(use_tc_tiling_on_sc=False)


@functools.partial(
    pl.kernel,
    out_type=jax.ShapeDtypeStruct((2, NP), jnp.float32),
    mesh=_mesh,
    scratch_types=[
        pltpu.VMEM((CH, 128), jnp.int32),
        pltpu.VMEM((128,), jnp.float32),
        pltpu.VMEM((RPT,), jnp.float32),
        pltpu.VMEM_SHARED((NP,), jnp.float32),
    ],
    compiler_params=_sc_params,
)
def _sc_deg(dst_hbm, deg_out, dstbuf, ones, zbuf, deg_sh):
    c = lax.axis_index("c")
    s = lax.axis_index("s")

    @pl.loop(0, 128, step=16)
    def _(i):
        ones[pl.ds(i, 16)] = jnp.full((16,), 1.0, jnp.float32)

    @pl.loop(0, RPT, step=16)
    def _(i):
        zbuf[pl.ds(i, 16)] = jnp.zeros((16,), jnp.float32)

    pltpu.sync_copy(zbuf, deg_sh.at[pl.ds(s * RPT, RPT)])
    plsc.subcore_barrier()

    pltpu.sync_copy(dst_hbm.at[c, s], dstbuf)

    @pl.loop(0, CH)
    def _(j):
        pltpu.sync_copy(ones, deg_sh.at[dstbuf.at[j]], add=True)

    plsc.subcore_barrier()
    pltpu.sync_copy(deg_sh.at[pl.ds(s * RPT, RPT)],
                    deg_out.at[c, pl.ds(s * RPT, RPT)])


@functools.partial(
    pl.kernel,
    out_type=jax.ShapeDtypeStruct((2, NP, H), jnp.float32),
    mesh=_mesh,
    scratch_types=[
        pltpu.VMEM((CH, 128), jnp.int32),
        pltpu.VMEM((CH, 128), jnp.int32),
        [pltpu.VMEM((128, H), jnp.float32) for _ in range(NB)],
        [pltpu.SemaphoreType.DMA for _ in range(NB)],
        [pltpu.SemaphoreType.DMA for _ in range(NB)],
        pltpu.VMEM_SHARED((NP, H), jnp.float32),
    ],
    compiler_params=_sc_params,
)
def _sc_agg(scaled_hbm, src_hbm, dst_hbm, acc_out,
            srcbuf, dstbuf, bufs, gsems, ssems, acc_sh):
    c = lax.axis_index("c")
    s = lax.axis_index("s")

    @pl.loop(0, 128)
    def _(r):
        @pl.loop(0, H, step=16)
        def _(k):
            bufs[0][r, pl.ds(k, 16)] = jnp.zeros((16,), jnp.float32)

    @pl.loop(0, RPT // 128)
    def _(k):
        pltpu.sync_copy(bufs[0], acc_sh.at[pl.ds(s * RPT + k * 128, 128)])

    plsc.subcore_barrier()

    pltpu.sync_copy(src_hbm.at[c, s], srcbuf)
    pltpu.sync_copy(dst_hbm.at[c, s], dstbuf)

    def gather(j, k):
        pltpu.async_copy(scaled_hbm.at[srcbuf.at[j]], bufs[k], gsems[k])

    def gwait(j, k):
        pltpu.make_async_copy(scaled_hbm.at[srcbuf.at[j]], bufs[k],
                              gsems[k]).wait()

    def scat(j, k):
        pltpu.async_copy(bufs[k], acc_sh.at[dstbuf.at[j]], ssems[k], add=True)

    def swait(j, k):
        pltpu.make_async_copy(bufs[k], acc_sh.at[dstbuf.at[j]],
                              ssems[k]).wait()

    for k in range(NB):
        gather(k, k)

    @pl.loop(0, SCH, step=NB)
    def _(j):
        for k in range(NB):
            gwait(j + k, k)
            scat(j + k, k)
            swait(j + k, k)
            gather(j + NB + k, k)

    for k in range(NB):
        gwait(SCH + k, k)

    plsc.subcore_barrier()

    @pl.loop(0, RPT // 128)
    def _(k):
        off = s * RPT + k * 128
        pltpu.sync_copy(acc_sh.at[pl.ds(off, 128)],
                        acc_out.at[c, pl.ds(off, 128)])


def _tc_scale_body(x_ref, w_ref, deg_ref, o_ref):
    h = jnp.dot(x_ref[...], w_ref[...], preferred_element_type=jnp.float32)
    dinv = lax.rsqrt(jnp.maximum(deg_ref[...], 1.0))
    o_ref[...] = h * dinv


def _tc_scale(xcat, w, degcol):
    blk = 1024
    grid = (2 * NP // blk,)
    return pl.pallas_call(
        _tc_scale_body,
        grid=grid,
        in_specs=[
            pl.BlockSpec((blk, D), lambda i: (i, 0)),
            pl.BlockSpec((D, H), lambda i: (0, 0)),
            pl.BlockSpec((blk, 1), lambda i: (i, 0)),
        ],
        out_specs=pl.BlockSpec((blk, H), lambda i: (i, 0)),
        out_shape=jax.ShapeDtypeStruct((2 * NP, H), jnp.float32),
    )(xcat, w, degcol)


def _tc_head_body(acc_ref, deg_ref, batch_ref, bg_ref,
                  w1_ref, b1_ref, w2_ref, b2_ref, w3_ref, b3_ref,
                  w4_ref, b4_ref, o_ref):
    means = []
    for g in range(2):
        accg = acc_ref[g]
        degg = deg_ref[g]
        dinv = lax.rsqrt(jnp.maximum(degg, 1.0))
        outg = jnp.maximum(accg * dinv + bg_ref[...], 0.0)
        batchg = batch_ref[g]
        iot = lax.broadcasted_iota(jnp.int32, (G, NP), 0)
        oh = (iot == batchg).astype(jnp.float32)
        sums = jnp.dot(oh, outg, preferred_element_type=jnp.float32)
        cnts = jnp.sum(oh, axis=1, keepdims=True)
        means.append(sums / jnp.maximum(cnts, 1.0))
    z = jnp.concatenate(means, axis=1)
    z = jnp.maximum(jnp.dot(z, w1_ref[...],
                            preferred_element_type=jnp.float32) + b1_ref[...], 0.0)
    z = jnp.maximum(jnp.dot(z, w2_ref[...],
                            preferred_element_type=jnp.float32) + b2_ref[...], 0.0)
    z = jnp.maximum(jnp.dot(z, w3_ref[...],
                            preferred_element_type=jnp.float32) + b3_ref[...], 0.0)
    z = jnp.dot(z, w4_ref[...], preferred_element_type=jnp.float32) + b4_ref[...]
    o_ref[...] = 1.0 / (1.0 + jnp.exp(-z))


def _tc_head(acc3, deg3, batch3, bg, w1, b1, w2, b2, w3, b3, w4, b4):
    return pl.pallas_call(
        _tc_head_body,
        out_shape=jax.ShapeDtypeStruct((G, 1), jnp.float32),
    )(acc3, deg3, batch3, bg, w1, b1, w2, b2, w3, b3, w4, b4)


def kernel(x_1, edge_index_1, x_1_batch, x_2, edge_index_2, x_2_batch,
           W_gcn, b_gcn, l1_w, l1_b, l2_w, l2_b, l3_w, l3_b, l4_w, l4_b):
    loop = jnp.arange(N, dtype=jnp.int32)

    def prep(ei, g):
        src = jnp.concatenate([ei[0], loop])
        dst = jnp.concatenate([ei[1], loop])
        src = jnp.pad(src, (0, EP - NE)) + g * NP
        dst = jnp.pad(dst, (0, EP - NE), constant_values=N)
        return src.reshape(16, CH, 128), dst.reshape(16, CH, 128)

    s1, d1 = prep(edge_index_1, 0)
    s2, d2 = prep(edge_index_2, 1)
    srccat = jnp.stack([s1, s2])
    dstcat = jnp.stack([d1, d2])

    xcat = jnp.concatenate([
        jnp.pad(x_1, ((0, NP - N), (0, 0))),
        jnp.pad(x_2, ((0, NP - N), (0, 0))),
    ])

    deg = _sc_deg(dstcat)
    scaled = _tc_scale(xcat, W_gcn, deg.reshape(2 * NP, 1))
    acc = _sc_agg(scaled, srccat, dstcat)

    batchcat = jnp.stack([
        jnp.pad(x_1_batch, (0, NP - N), constant_values=G),
        jnp.pad(x_2_batch, (0, NP - N), constant_values=G),
    ]).reshape(2, 1, NP)

    return _tc_head(
        acc, deg.reshape(2, NP, 1), batchcat,
        b_gcn.reshape(1, H),
        l1_w, l1_b.reshape(1, 64),
        l2_w, l2_b.reshape(1, 32),
        l3_w, l3_b.reshape(1, 16),
        l4_w, l4_b.reshape(1, 1),
    )


# speedup vs baseline: 2.0593x; 1.4113x over previous
import functools

import jax
import jax.numpy as jnp
from jax import lax
from jax.experimental import pallas as pl
from jax.experimental.pallas import tpu as pltpu
from jax.experimental.pallas import tpu_sc as plsc

N = 10000
E = 320000
D = 128
H = 64
G = 64

NP = 10240
RPT = NP // 16
NE = E + N
NB = 2
SCH = 162
CH = SCH + NB
EPT = CH * 128
EP = 16 * EPT

_mesh = plsc.VectorSubcoreMesh(core_axis_name="c", subcore_axis_name="s")
_sc_params = pltpu.CompilerParams(use_tc_tiling_on_sc=False)


@functools.partial(
    pl.kernel,
    out_type=jax.ShapeDtypeStruct((2, NP), jnp.float32),
    mesh=_mesh,
    scratch_types=[
        pltpu.VMEM((CH, 128), jnp.int32),
        pltpu.VMEM((128,), jnp.float32),
        pltpu.VMEM((RPT,), jnp.float32),
        pltpu.VMEM_SHARED((NP,), jnp.float32),
    ],
    compiler_params=_sc_params,
)
def _sc_deg(dst_hbm, deg_out, dstbuf, ones, zbuf, deg_sh):
    c = lax.axis_index("c")
    s = lax.axis_index("s")

    @pl.loop(0, 128, step=16)
    def _(i):
        ones[pl.ds(i, 16)] = jnp.full((16,), 1.0, jnp.float32)

    @pl.loop(0, RPT, step=16)
    def _(i):
        zbuf[pl.ds(i, 16)] = jnp.zeros((16,), jnp.float32)

    pltpu.sync_copy(zbuf, deg_sh.at[pl.ds(s * RPT, RPT)])
    plsc.subcore_barrier()

    pltpu.sync_copy(dst_hbm.at[c, s], dstbuf)

    @pl.loop(0, CH)
    def _(j):
        pltpu.sync_copy(ones, deg_sh.at[dstbuf.at[j]], add=True)

    plsc.subcore_barrier()
    pltpu.sync_copy(deg_sh.at[pl.ds(s * RPT, RPT)],
                    deg_out.at[c, pl.ds(s * RPT, RPT)])


@functools.partial(
    pl.kernel,
    out_type=jax.ShapeDtypeStruct((2, NP, H), jnp.bfloat16),
    mesh=_mesh,
    scratch_types=[
        pltpu.VMEM((CH, 128), jnp.int32),
        pltpu.VMEM((CH, 128), jnp.int32),
        [pltpu.VMEM((128, H), jnp.bfloat16) for _ in range(NB)],
        [pltpu.SemaphoreType.DMA for _ in range(NB)],
        [pltpu.SemaphoreType.DMA for _ in range(NB)],
        pltpu.VMEM_SHARED((NP, H), jnp.bfloat16),
    ],
    compiler_params=_sc_params,
)
def _sc_agg(scaled_hbm, src_hbm, dst_hbm, acc_out,
            srcbuf, dstbuf, bufs, gsems, ssems, acc_sh):
    c = lax.axis_index("c")
    s = lax.axis_index("s")

    @pl.loop(0, 128)
    def _(r):
        @pl.loop(0, H, step=32)
        def _(k):
            bufs[0][r, pl.ds(k, 32)] = jnp.zeros((32,), jnp.bfloat16)

    @pl.loop(0, RPT // 128)
    def _(k):
        pltpu.sync_copy(bufs[0], acc_sh.at[pl.ds(s * RPT + k * 128, 128)])

    plsc.subcore_barrier()

    pltpu.sync_copy(src_hbm.at[c, s], srcbuf)
    pltpu.sync_copy(dst_hbm.at[c, s], dstbuf)

    def gather(j, k):
        pltpu.async_copy(scaled_hbm.at[srcbuf.at[j]], bufs[k], gsems[k])

    def gwait(j, k):
        pltpu.make_async_copy(scaled_hbm.at[srcbuf.at[j]], bufs[k],
                              gsems[k]).wait()

    def scat(j, k):
        pltpu.async_copy(bufs[k], acc_sh.at[dstbuf.at[j]], ssems[k], add=True)

    def swait(j, k):
        pltpu.make_async_copy(bufs[k], acc_sh.at[dstbuf.at[j]],
                              ssems[k]).wait()

    for k in range(NB):
        gather(k, k)

    @pl.loop(0, SCH, step=NB)
    def _(j):
        for k in range(NB):
            gwait(j + k, k)
            scat(j + k, k)
            swait(j + k, k)
            gather(j + NB + k, k)

    for k in range(NB):
        gwait(SCH + k, k)

    plsc.subcore_barrier()

    @pl.loop(0, RPT // 128)
    def _(k):
        off = s * RPT + k * 128
        pltpu.sync_copy(acc_sh.at[pl.ds(off, 128)],
                        acc_out.at[c, pl.ds(off, 128)])


def _tc_scale_body(x_ref, w_ref, deg_ref, o_ref):
    h = jnp.dot(x_ref[...], w_ref[...], preferred_element_type=jnp.float32)
    dinv = lax.rsqrt(jnp.maximum(deg_ref[...], 1.0))
    o_ref[...] = (h * dinv).astype(jnp.bfloat16)


def _tc_scale(xcat, w, degcol):
    blk = 1024
    grid = (2 * NP // blk,)
    return pl.pallas_call(
        _tc_scale_body,
        grid=grid,
        in_specs=[
            pl.BlockSpec((blk, D), lambda i: (i, 0)),
            pl.BlockSpec((D, H), lambda i: (0, 0)),
            pl.BlockSpec((blk, 1), lambda i: (i, 0)),
        ],
        out_specs=pl.BlockSpec((blk, H), lambda i: (i, 0)),
        out_shape=jax.ShapeDtypeStruct((2 * NP, H), jnp.bfloat16),
    )(xcat, w, degcol)


def _tc_head_body(acc_ref, deg_ref, batch_ref, bg_ref,
                  w1_ref, b1_ref, w2_ref, b2_ref, w3_ref, b3_ref,
                  w4_ref, b4_ref, o_ref):
    means = []
    for g in range(2):
        accg = acc_ref[g].astype(jnp.float32)
        degg = deg_ref[g]
        dinv = lax.rsqrt(jnp.maximum(degg, 1.0))
        outg = jnp.maximum(accg * dinv + bg_ref[...], 0.0)
        batchg = batch_ref[g]
        iot = lax.broadcasted_iota(jnp.int32, (G, NP), 0)
        oh = (iot == batchg).astype(jnp.float32)
        sums = jnp.dot(oh, outg, preferred_element_type=jnp.float32)
        cnts = jnp.sum(oh, axis=1, keepdims=True)
        means.append(sums / jnp.maximum(cnts, 1.0))
    z = jnp.concatenate(means, axis=1)
    z = jnp.maximum(jnp.dot(z, w1_ref[...],
                            preferred_element_type=jnp.float32) + b1_ref[...], 0.0)
    z = jnp.maximum(jnp.dot(z, w2_ref[...],
                            preferred_element_type=jnp.float32) + b2_ref[...], 0.0)
    z = jnp.maximum(jnp.dot(z, w3_ref[...],
                            preferred_element_type=jnp.float32) + b3_ref[...], 0.0)
    z = jnp.dot(z, w4_ref[...], preferred_element_type=jnp.float32) + b4_ref[...]
    o_ref[...] = 1.0 / (1.0 + jnp.exp(-z))


def _tc_head(acc3, deg3, batch3, bg, w1, b1, w2, b2, w3, b3, w4, b4):
    return pl.pallas_call(
        _tc_head_body,
        out_shape=jax.ShapeDtypeStruct((G, 1), jnp.float32),
    )(acc3, deg3, batch3, bg, w1, b1, w2, b2, w3, b3, w4, b4)


def kernel(x_1, edge_index_1, x_1_batch, x_2, edge_index_2, x_2_batch,
           W_gcn, b_gcn, l1_w, l1_b, l2_w, l2_b, l3_w, l3_b, l4_w, l4_b):
    loop = jnp.arange(N, dtype=jnp.int32)

    def prep(ei, g):
        src = jnp.concatenate([ei[0], loop])
        dst = jnp.concatenate([ei[1], loop])
        src = jnp.pad(src, (0, EP - NE)) + g * NP
        dst = jnp.pad(dst, (0, EP - NE), constant_values=N)
        return src.reshape(16, CH, 128), dst.reshape(16, CH, 128)

    s1, d1 = prep(edge_index_1, 0)
    s2, d2 = prep(edge_index_2, 1)
    srccat = jnp.stack([s1, s2])
    dstcat = jnp.stack([d1, d2])

    xcat = jnp.concatenate([
        jnp.pad(x_1, ((0, NP - N), (0, 0))),
        jnp.pad(x_2, ((0, NP - N), (0, 0))),
    ])

    deg = _sc_deg(dstcat)
    scaled = _tc_scale(xcat, W_gcn, deg.reshape(2 * NP, 1))
    acc = _sc_agg(scaled, srccat, dstcat)

    batchcat = jnp.stack([
        jnp.pad(x_1_batch, (0, NP - N), constant_values=G),
        jnp.pad(x_2_batch, (0, NP - N), constant_values=G),
    ]).reshape(2, 1, NP)

    return _tc_head(
        acc, deg.reshape(2, NP, 1), batchcat,
        b_gcn.reshape(1, H),
        l1_w, l1_b.reshape(1, 64),
        l2_w, l2_b.reshape(1, 32),
        l3_w, l3_b.reshape(1, 16),
        l4_w, l4_b.reshape(1, 1),
    )
